# baseline (speedup 1.0000x reference)
import jax
import jax.numpy as jnp
from jax import lax
from jax.experimental import pallas as pl
from jax.experimental.pallas import tpu as pltpu
from jax.experimental.pallas import tpu_sc as plsc

_DIM = 64
_LANE = 128
_K = 8
_NW = 32


def _gather_body(table_hbm, idx_hbm, out_hbm, idx_v, rows_v, gsem):
    nrow = idx_hbm.shape[0]
    rows_per_w = nrow // _NW
    nchunk = rows_per_w // _K
    wid = lax.axis_index("s") * 2 + lax.axis_index("c")
    row0 = wid * rows_per_w

    def body(c, carry):
        base = row0 + c * _K
        pltpu.sync_copy(idx_hbm.at[pl.ds(base, _K)], idx_v)
        copies = [
            pltpu.async_copy(table_hbm.at[idx_v.at[j]], rows_v.at[j], gsem)
            for j in range(_K)
        ]
        for cp in copies:
            cp.wait()
        pltpu.sync_copy(rows_v, out_hbm.at[pl.ds(base, _K)])
        return carry

    lax.fori_loop(0, nchunk, body, 0)


def kernel(inputs, table):
    seq, batch = inputs.shape
    n = seq * batch
    nrow = n // _LANE
    idx = inputs.reshape(nrow, _LANE)
    mesh = plsc.VectorSubcoreMesh(core_axis_name="c", subcore_axis_name="s")
    out = pl.kernel(
        _gather_body,
        out_type=jax.ShapeDtypeStruct((nrow, _LANE, _DIM), jnp.float32),
        mesh=mesh,
        compiler_params=pltpu.CompilerParams(use_tc_tiling_on_sc=False),
        scratch_types=[
            pltpu.VMEM((_K, _LANE), jnp.int32),
            pltpu.VMEM((_K, _LANE, _DIM), jnp.float32),
            pltpu.SemaphoreType.DMA,
        ],
    )(table, idx)
    return out.reshape(seq, batch, _DIM)
